# baseline (speedup 1.0000x reference)
import functools

import jax
import jax.numpy as jnp
from jax import lax
from jax.experimental import pallas as pl
from jax.experimental.pallas import tpu as pltpu
from jax.experimental.pallas import tpu_sc as plsc

BATCH = 16384
EMB = 64
LANES = 16
NUM_CORES = 2
NUM_SUBCORES = 16
NUM_WORKERS = NUM_CORES * NUM_SUBCORES
BPW = BATCH // NUM_WORKERS
CHUNK = 128
NCHUNK = BPW // CHUNK
NGROUP = BPW // LANES
MAX_ITEM_ROWS = 1000000 // LANES


def _lane_perm(x, idx):
    dnums = lax.GatherDimensionNumbers(
        offset_dims=(), collapsed_slice_dims=(0,), start_index_map=(0,))
    return lax.gather(x, idx[:, None], dnums, slice_sizes=(1,),
                      mode=lax.GatherScatterMode.PROMISE_IN_BOUNDS)


def _body(uid_hbm, iid_hbm, utab_hbm, itab_hbm, ibias_hbm, out_hbm,
          uidx, iidx, iidx_flat, bidx, u_v, v_v, brows, out_v, sem):
    wid = lax.axis_index("s") * NUM_CORES + lax.axis_index("c")
    base = wid * BPW

    for j in range(NCHUNK):
        pltpu.sync_copy(uid_hbm.at[pl.ds(base + j * CHUNK, CHUNK)], uidx.at[j])
        pltpu.sync_copy(iid_hbm.at[pl.ds(base + j * CHUNK, CHUNK)], iidx.at[j])
    pltpu.sync_copy(iid_hbm.at[pl.ds(base, BPW)], iidx_flat)

    for j in range(NCHUNK):
        for o in range(CHUNK // LANES):
            sl = pl.ds(o * LANES, LANES)
            bidx[j, sl] = jnp.right_shift(iidx[j, sl], 4)

    copies = []
    for j in range(NCHUNK):
        sl = pl.ds(j * CHUNK, CHUNK)
        copies.append(pltpu.async_copy(utab_hbm.at[uidx.at[j]], u_v.at[sl], sem))
        copies.append(pltpu.async_copy(itab_hbm.at[iidx.at[j]], v_v.at[sl], sem))
        copies.append(pltpu.async_copy(ibias_hbm.at[bidx.at[j]], brows.at[sl], sem))
    for c in copies:
        c.wait()

    iota16 = lax.iota(jnp.int32, LANES)

    def group_body(g, carry):
        sl = pl.ds(g * LANES, LANES)
        res = jnp.zeros((LANES,), jnp.float32)
        lanes_vec = iidx_flat[sl] & (LANES - 1)
        for j in range(LANES):
            b = g * LANES + j
            acc = u_v[b, pl.ds(0, LANES)] * v_v[b, pl.ds(0, LANES)]
            for k in range(1, EMB // LANES):
                acc = acc + u_v[b, pl.ds(k * LANES, LANES)] * v_v[b, pl.ds(k * LANES, LANES)]
            lane = lanes_vec[j]
            acc = acc + jnp.where(iota16 == lane, brows[b, pl.ds(0, LANES)], 0.0)
            for step in (1, 2, 4, 8):
                acc = acc + _lane_perm(acc, iota16 ^ step)
            res = jnp.where(iota16 == j, acc, res)
        out_v[sl] = res
        return carry

    lax.fori_loop(0, NGROUP, group_body, 0)

    pltpu.sync_copy(out_v, out_hbm.at[pl.ds(base, BPW)])


_cf_kernel = functools.partial(
    pl.kernel,
    out_type=jax.ShapeDtypeStruct((BATCH,), jnp.float32),
    scratch_types=[
        pltpu.VMEM((NCHUNK, CHUNK), jnp.int32),
        pltpu.VMEM((NCHUNK, CHUNK), jnp.int32),
        pltpu.VMEM((BPW,), jnp.int32),
        pltpu.VMEM((NCHUNK, CHUNK), jnp.int32),
        pltpu.VMEM((BPW, EMB), jnp.float32),
        pltpu.VMEM((BPW, EMB), jnp.float32),
        pltpu.VMEM((BPW, LANES), jnp.float32),
        pltpu.VMEM((BPW,), jnp.float32),
        pltpu.SemaphoreType.DMA,
    ],
    mesh=plsc.VectorSubcoreMesh(core_axis_name="c", subcore_axis_name="s"),
    compiler_params=pltpu.CompilerParams(use_tc_tiling_on_sc=False),
)(_body)


@jax.jit
def kernel(user_ids, item_ids, user_table, item_table, item_bias):
    return _cf_kernel(user_ids.astype(jnp.int32), item_ids.astype(jnp.int32),
                      user_table, item_table,
                      item_bias.reshape(MAX_ITEM_ROWS, LANES))


# speedup vs baseline: 1.4398x; 1.4398x over previous
import functools

import jax
import jax.numpy as jnp
from jax import lax
from jax.experimental import pallas as pl
from jax.experimental.pallas import tpu as pltpu
from jax.experimental.pallas import tpu_sc as plsc

BATCH = 16384
EMB = 64
LANES = 16
NUM_CORES = 2
NUM_SUBCORES = 16
NUM_WORKERS = NUM_CORES * NUM_SUBCORES
BPW = BATCH // NUM_WORKERS
HALF = BPW // 2
NGROUP = HALF // LANES
MAX_ITEM_ROWS = 1000000 // LANES


def _lane_perm(x, idx):
    dnums = lax.GatherDimensionNumbers(
        offset_dims=(), collapsed_slice_dims=(0,), start_index_map=(0,))
    return lax.gather(x, idx[:, None], dnums, slice_sizes=(1,),
                      mode=lax.GatherScatterMode.PROMISE_IN_BOUNDS)


def _body(uid_hbm, iid_hbm, utab_hbm, itab_hbm, ibias_hbm, out_hbm,
          uidx, iidx, u_v, v_v, brows, out_v, sem):
    wid = lax.axis_index("s") * NUM_CORES + lax.axis_index("c")
    base = wid * BPW

    pltpu.sync_copy(uid_hbm.at[pl.ds(base, BPW)], uidx)
    pltpu.sync_copy(iid_hbm.at[pl.ds(base, BPW)], iidx)

    iota16 = lax.iota(jnp.int32, LANES)

    for p in range(2):
        def fetch_body(g, carry):
            sl = pl.ds(p * HALF + g * LANES, LANES)
            uvec = uidx[sl]
            ivec = iidx[sl]
            bvec = jnp.right_shift(ivec, 4)
            for j in range(LANES):
                r = g * LANES + j
                pltpu.async_copy(utab_hbm.at[uvec[j]], u_v.at[r], sem)
                pltpu.async_copy(itab_hbm.at[ivec[j]], v_v.at[r], sem)
                pltpu.async_copy(ibias_hbm.at[bvec[j]], brows.at[r], sem)
            return carry

        lax.fori_loop(0, NGROUP, fetch_body, 0)

        pltpu.make_async_copy(utab_hbm.at[pl.ds(0, HALF)], u_v, sem).wait()
        pltpu.make_async_copy(itab_hbm.at[pl.ds(0, HALF)], v_v, sem).wait()
        pltpu.make_async_copy(ibias_hbm.at[pl.ds(0, HALF)], brows, sem).wait()

        def group_body(g, carry):
            sl = pl.ds(p * HALF + g * LANES, LANES)
            res = jnp.zeros((LANES,), jnp.float32)
            lanes_vec = iidx[sl] & (LANES - 1)
            for j in range(LANES):
                r = g * LANES + j
                acc = u_v[r, pl.ds(0, LANES)] * v_v[r, pl.ds(0, LANES)]
                for k in range(1, EMB // LANES):
                    acc = acc + (u_v[r, pl.ds(k * LANES, LANES)]
                                 * v_v[r, pl.ds(k * LANES, LANES)])
                lane = lanes_vec[j]
                acc = acc + jnp.where(iota16 == lane,
                                      brows[r, pl.ds(0, LANES)], 0.0)
                for step in (1, 2, 4, 8):
                    acc = acc + _lane_perm(acc, iota16 ^ step)
                res = jnp.where(iota16 == j, acc, res)
            out_v[sl] = res
            return carry

        lax.fori_loop(0, NGROUP, group_body, 0)

    pltpu.sync_copy(out_v, out_hbm.at[pl.ds(base, BPW)])


_cf_kernel = functools.partial(
    pl.kernel,
    out_type=jax.ShapeDtypeStruct((BATCH,), jnp.float32),
    scratch_types=[
        pltpu.VMEM((BPW,), jnp.int32),
        pltpu.VMEM((BPW,), jnp.int32),
        pltpu.VMEM((HALF, EMB), jnp.float32),
        pltpu.VMEM((HALF, EMB), jnp.float32),
        pltpu.VMEM((HALF, LANES), jnp.float32),
        pltpu.VMEM((BPW,), jnp.float32),
        pltpu.SemaphoreType.DMA,
    ],
    mesh=plsc.VectorSubcoreMesh(core_axis_name="c", subcore_axis_name="s"),
)(_body)


@jax.jit
def kernel(user_ids, item_ids, user_table, item_table, item_bias):
    return _cf_kernel(user_ids.astype(jnp.int32), item_ids.astype(jnp.int32),
                      user_table, item_table,
                      item_bias.reshape(MAX_ITEM_ROWS, LANES))
